# baseline (speedup 1.0000x reference)
import functools

import jax
import jax.numpy as jnp
from jax.experimental import pallas as pl
from jax.experimental.pallas import tpu as pltpu


def _gcn_layer_kernel(adj_hbm, v_ref, w_ref, b_ref, out_ref, buf, sems,
                      *, relu, bm, nchunk):
    i = pl.program_id(0)
    nsteps = pl.num_programs(0)
    ck = bm // nchunk

    def issue(step, slot):
        base = step * bm
        for c in range(nchunk):
            pltpu.make_async_copy(
                adj_hbm.at[pl.ds(base + c * ck, ck), :],
                buf.at[slot, pl.ds(c * ck, ck), :],
                sems.at[slot],
            ).start()

    @pl.when(i == 0)
    def _():
        issue(0, 0)

    @pl.when(i + 1 < nsteps)
    def _():
        issue(i + 1, (i + 1) % 2)

    slot = i % 2
    for c in range(nchunk):
        pltpu.make_async_copy(
            adj_hbm.at[pl.ds(c * ck, ck), :],
            buf.at[slot, pl.ds(c * ck, ck), :],
            sems.at[slot],
        ).wait()

    a16 = buf[slot].astype(jnp.bfloat16)
    t = jnp.dot(a16, v_ref[...], preferred_element_type=jnp.float32)
    t = jnp.dot(t.astype(jnp.bfloat16), w_ref[...].astype(jnp.bfloat16),
                preferred_element_type=jnp.float32) + b_ref[...]
    if relu:
        t = jnp.maximum(t, 0.0)
    out_ref[...] = t.astype(out_ref.dtype)


def _gcn_layer(adj, v, w, b, *, relu, out_dtype, bm, nchunk):
    n, k = adj.shape
    d = w.shape[1]
    return pl.pallas_call(
        functools.partial(_gcn_layer_kernel, relu=relu, bm=bm, nchunk=nchunk),
        grid=(n // bm,),
        in_specs=[
            pl.BlockSpec(memory_space=pl.ANY),
            pl.BlockSpec((k, v.shape[1]), lambda i: (0, 0)),
            pl.BlockSpec(w.shape, lambda i: (0, 0)),
            pl.BlockSpec((1, d), lambda i: (0, 0)),
        ],
        out_specs=pl.BlockSpec((bm, d), lambda i: (i, 0)),
        out_shape=jax.ShapeDtypeStruct((n, d), out_dtype),
        scratch_shapes=[
            pltpu.VMEM((2, bm, k), jnp.float32),
            pltpu.SemaphoreType.DMA((2,)),
        ],
    )(adj, v, w, b.reshape(1, d))


def kernel(x, adj, W1, b1, W2, b2):
    x16 = x.astype(jnp.bfloat16)
    h16 = _gcn_layer(adj, x16, W1, b1, relu=True, out_dtype=jnp.bfloat16,
                     bm=400, nchunk=10)
    return _gcn_layer(adj, h16, W2, b2, relu=False, out_dtype=jnp.float32,
                      bm=400, nchunk=10)


# speedup vs baseline: 1.0025x; 1.0025x over previous
import functools

import jax
import jax.numpy as jnp
from jax.experimental import pallas as pl
from jax.experimental.pallas import tpu as pltpu


def _gcn_kernel(adj_hbm, x_ref, w_ref, b_ref, out_ref, buf, h_ref, sems,
                *, bm, nchunk):
    p = pl.program_id(0)
    i = pl.program_id(1)
    nsteps = pl.num_programs(1)
    g = p * nsteps + i
    ck = bm // nchunk

    def issue(step, slot):
        base = (step % nsteps) * bm
        for c in range(nchunk):
            pltpu.make_async_copy(
                adj_hbm.at[pl.ds(base + c * ck, ck), :],
                buf.at[slot, pl.ds(c * ck, ck), :],
                sems.at[slot],
            ).start()

    @pl.when(g == 0)
    def _():
        issue(0, 0)

    @pl.when(g + 1 < 2 * nsteps)
    def _():
        issue(g + 1, (g + 1) % 2)

    slot = g % 2
    for c in range(nchunk):
        pltpu.make_async_copy(
            adj_hbm.at[pl.ds(c * ck, ck), :],
            buf.at[slot, pl.ds(c * ck, ck), :],
            sems.at[slot],
        ).wait()

    a16 = buf[slot].astype(jnp.bfloat16)

    @pl.when(p == 0)
    def _():
        t = jnp.dot(a16, x_ref[...], preferred_element_type=jnp.float32)
        t = jnp.dot(t.astype(jnp.bfloat16), w_ref[0],
                    preferred_element_type=jnp.float32) + b_ref[0]
        h_ref[pl.ds(i * bm, bm), :] = jnp.maximum(t, 0.0).astype(jnp.bfloat16)

    @pl.when(p == 1)
    def _():
        t = jnp.dot(a16, h_ref[...], preferred_element_type=jnp.float32)
        t = jnp.dot(t.astype(jnp.bfloat16), w_ref[1],
                    preferred_element_type=jnp.float32) + b_ref[1]
        out_ref[...] = t


def kernel(x, adj, W1, b1, W2, b2):
    n, k = adj.shape
    d = W1.shape[1]
    bm, nchunk = 400, 10
    x16 = x.astype(jnp.bfloat16)
    w = jnp.stack([W1.astype(jnp.bfloat16), W2.astype(jnp.bfloat16)])
    b = jnp.stack([b1, b2]).reshape(2, 1, d)
    return pl.pallas_call(
        functools.partial(_gcn_kernel, bm=bm, nchunk=nchunk),
        grid=(2, n // bm),
        in_specs=[
            pl.BlockSpec(memory_space=pl.ANY),
            pl.BlockSpec((k, d), lambda p, i: (0, 0)),
            pl.BlockSpec((2, d, d), lambda p, i: (0, 0, 0)),
            pl.BlockSpec((2, 1, d), lambda p, i: (0, 0, 0)),
        ],
        out_specs=pl.BlockSpec((bm, d), lambda p, i: (p * i, 0)),
        out_shape=jax.ShapeDtypeStruct((n, d), jnp.float32),
        scratch_shapes=[
            pltpu.VMEM((2, bm, k), jnp.float32),
            pltpu.VMEM((n, d), jnp.bfloat16),
            pltpu.SemaphoreType.DMA((2,)),
        ],
    )(adj, x16, w, b)
